# Initial kernel scaffold; baseline (speedup 1.0000x reference)
import jax
import jax.numpy as jnp
from jax.experimental import pallas as pl


def kernel(node_feats, batch, W_attend, gru_w_ih, gru_w_hh, gru_b_ih, gru_b_hh):
    raise NotImplementedError("write your pallas kernel here")


# speedup vs baseline: 12.9640x; 12.9640x over previous
import jax
import jax.numpy as jnp
from jax.experimental import pallas as pl
from jax.experimental.pallas import tpu as pltpu

_N = 50000
_H = 256
_B = 512
_T = 2
_NB = 2000
_G = _N // _NB


def _onehot_t(batch_ref):
    b2 = batch_ref[0]
    iota = jax.lax.broadcasted_iota(jnp.int32, (_B, _NB), 0)
    return iota == b2


def _k1_body(node_ref, w_ref, batch_ref, att_ref, counts_ref, sum0_ref):
    i = pl.program_id(0)
    node = node_ref[...]
    att_ref[...] = jax.lax.dot_general(
        node, w_ref[...], (((1,), (1,)), ((), ())),
        preferred_element_type=jnp.float32)
    pt = _onehot_t(batch_ref).astype(jnp.float32)
    cnt = jnp.sum(pt, axis=1, keepdims=True)
    s0 = jax.lax.dot_general(pt, node, (((1,), (0,)), ((), ())),
                             preferred_element_type=jnp.float32)

    @pl.when(i == 0)
    def _():
        counts_ref[...] = jnp.zeros_like(counts_ref)
        sum0_ref[...] = jnp.zeros_like(sum0_ref)

    counts_ref[...] += cnt
    sum0_ref[...] += s0


def _k2_body(sum0_ref, counts_ref, gf_ref):
    gf_ref[...] = sum0_ref[...] / jnp.maximum(counts_ref[...], 1.0)


def _ka_body(att_ref, gf_ref, batch_ref, scores_ref, m_ref):
    i = pl.program_id(0)
    st = jax.lax.dot_general(gf_ref[...], att_ref[...],
                             (((1,), (1,)), ((), ())),
                             preferred_element_type=jnp.float32)
    ptb = _onehot_t(batch_ref)
    scores_ref[0, 0, :] = jnp.sum(jnp.where(ptb, st, 0.0), axis=0)
    mt = jnp.max(jnp.where(ptb, st, -jnp.inf), axis=1, keepdims=True)

    @pl.when(i == 0)
    def _():
        m_ref[...] = jnp.full_like(m_ref, -jnp.inf)

    m_ref[...] = jnp.maximum(m_ref[...], mt)


def _kb_body(node_ref, batch_ref, scores_ref, m_ref, ssum_ref, ctx_ref):
    i = pl.program_id(0)
    ptb = _onehot_t(batch_ref)
    m = m_ref[...]
    mfix = jnp.where(jnp.isfinite(m), m, 0.0)
    msel = jnp.sum(jnp.where(ptb, mfix, 0.0), axis=0)
    e = jnp.exp(scores_ref[0, 0, :] - msel)
    pte = ptb.astype(jnp.float32) * e[None, :]
    ssum_t = jnp.sum(pte, axis=1, keepdims=True)
    ctx_t = jax.lax.dot_general(pte, node_ref[...],
                                (((1,), (0,)), ((), ())),
                                preferred_element_type=jnp.float32)

    @pl.when(i == 0)
    def _():
        ssum_ref[...] = jnp.zeros_like(ssum_ref)
        ctx_ref[...] = jnp.zeros_like(ctx_ref)

    ssum_ref[...] += ssum_t
    ctx_ref[...] += ctx_t


def _kc_body(ctx_ref, ssum_ref, gf_ref, wih_ref, whh_ref, bih_ref, bhh_ref,
             out_ref):
    ctx = ctx_ref[...] / (ssum_ref[...] + 1e-16)
    h = gf_ref[...]
    gi = jax.lax.dot_general(ctx, wih_ref[...], (((1,), (1,)), ((), ())),
                             preferred_element_type=jnp.float32)
    gi = gi + bih_ref[...][None, :]
    gh = jax.lax.dot_general(h, whh_ref[...], (((1,), (1,)), ((), ())),
                             preferred_element_type=jnp.float32)
    gh = gh + bhh_ref[...][None, :]
    r = jax.nn.sigmoid(gi[:, 0:_H] + gh[:, 0:_H])
    z = jax.nn.sigmoid(gi[:, _H:2 * _H] + gh[:, _H:2 * _H])
    n = jnp.tanh(gi[:, 2 * _H:] + r * gh[:, 2 * _H:])
    out_ref[...] = (1.0 - z) * n + z * h


def _full(shape):
    return pl.BlockSpec(shape, lambda *a: tuple(0 for _ in shape))


def kernel(node_feats, batch, W_attend, gru_w_ih, gru_w_hh, gru_b_ih,
           gru_b_hh):
    batch3 = batch.reshape(_G, 1, _NB)
    node_spec = pl.BlockSpec((_NB, _H), lambda i: (i, 0))
    batch_spec = pl.BlockSpec((1, 1, _NB), lambda i: (i, 0, 0))
    scores_spec = pl.BlockSpec((1, 1, _NB), lambda i: (i, 0, 0))

    att, counts, sum0 = pl.pallas_call(
        _k1_body,
        grid=(_G,),
        in_specs=[node_spec, _full((_H, _H)), batch_spec],
        out_specs=[node_spec, _full((_B, 1)), _full((_B, _H))],
        out_shape=[
            jax.ShapeDtypeStruct((_N, _H), jnp.float32),
            jax.ShapeDtypeStruct((_B, 1), jnp.float32),
            jax.ShapeDtypeStruct((_B, _H), jnp.float32),
        ],
    )(node_feats, W_attend, batch3)

    gf = pl.pallas_call(
        _k2_body,
        in_specs=[_full((_B, _H)), _full((_B, 1))],
        out_specs=_full((_B, _H)),
        out_shape=jax.ShapeDtypeStruct((_B, _H), jnp.float32),
    )(sum0, counts)

    for _ in range(_T):
        scores, m = pl.pallas_call(
            _ka_body,
            grid=(_G,),
            in_specs=[node_spec, _full((_B, _H)), batch_spec],
            out_specs=[scores_spec, _full((_B, 1))],
            out_shape=[
                jax.ShapeDtypeStruct((_G, 1, _NB), jnp.float32),
                jax.ShapeDtypeStruct((_B, 1), jnp.float32),
            ],
        )(att, gf, batch3)

        ssum, ctx = pl.pallas_call(
            _kb_body,
            grid=(_G,),
            in_specs=[node_spec, batch_spec, scores_spec, _full((_B, 1))],
            out_specs=[_full((_B, 1)), _full((_B, _H))],
            out_shape=[
                jax.ShapeDtypeStruct((_B, 1), jnp.float32),
                jax.ShapeDtypeStruct((_B, _H), jnp.float32),
            ],
        )(node_feats, batch3, scores, m)

        gf = pl.pallas_call(
            _kc_body,
            in_specs=[_full((_B, _H)), _full((_B, 1)), _full((_B, _H)),
                      _full((3 * _H, _H)), _full((3 * _H, _H)),
                      _full((3 * _H,)), _full((3 * _H,))],
            out_specs=_full((_B, _H)),
            out_shape=jax.ShapeDtypeStruct((_B, _H), jnp.float32),
        )(ctx, ssum, gf, gru_w_ih, gru_w_hh, gru_b_ih, gru_b_hh)

    return gf
